# baseline (device time: 1471268 ns/iter reference)
import jax
import jax.numpy as jnp
from jax import lax
from jax.experimental import pallas as pl
from jax.experimental.pallas import tpu as pltpu

N_DEV = 16
M, N = 8192, 4096
MC = M // N_DEV
H = MC // 2
Q = 4
HQ = H // Q
NSLOT = 4

_PLANE = {(0, 0): 0, (1, 0): 1, (1, 1): 2, (0, 1): 3}
_RING_COORDS = (
    [(0, 0, z) for z in range(4)]
    + [(1, 0, z) for z in range(3, -1, -1)]
    + [(1, 1, z) for z in range(4)]
    + [(0, 1, z) for z in range(3, -1, -1)]
)
_RING_MESH = [4 * z + _PLANE[(x, y)] for (x, y, z) in _RING_COORDS]
_MESH_TO_RING = [0] * N_DEV
for _r, _p in enumerate(_RING_MESH):
    _MESH_TO_RING[_p] = _r
_RIGHT = [_RING_MESH[(_MESH_TO_RING[p] + 1) % N_DEV] for p in range(N_DEV)]
_LEFT = [_RING_MESH[(_MESH_TO_RING[p] - 1) % N_DEV] for p in range(N_DEV)]

_F32 = jnp.float32
_LOGICAL = pl.DeviceIdType.LOGICAL


def _body(scal_ref, xbf, wbf, out_ref, accA, accB, locA, locB,
          copy_sems, rsA_send, rsA_recv, rsB_send, rsB_recv,
          agA_send, agA_recv, agB_send, agB_recv, creditA, creditB):
    my_ring = scal_ref[0]
    right = scal_ref[1]
    left = scal_ref[2]

    def cw(k):
        return lax.rem(my_ring - k + 2 * N_DEV, N_DEV)

    def ccw(k):
        return lax.rem(my_ring + k, N_DEV)

    def subA(slot, q):
        return accA.at[slot, pl.ds(q * HQ, HQ), :]

    def subB(slot, q):
        return accB.at[slot, pl.ds(q * HQ, HQ), :]

    def outA(c, q):
        return out_ref.at[pl.ds(c * MC + q * HQ, HQ), :]

    def outB(c, q):
        return out_ref.at[pl.ds(c * MC + H + q * HQ, HQ), :]

    def partial(c, half_off):
        xs = xbf[pl.ds(c * MC + half_off, H), :]
        return jnp.dot(xs, wbf[...], preferred_element_type=_F32)

    def rs_send(ring_sub, slot_src, slot_dst, sems_s, sems_r, s, q, dev):
        d = pltpu.make_async_remote_copy(
            src_ref=ring_sub(slot_src, q), dst_ref=ring_sub(slot_dst, q),
            send_sem=sems_s.at[s, q], recv_sem=sems_r.at[s, q],
            device_id=dev, device_id_type=_LOGICAL)
        d.start()
        return d

    barrier = pltpu.get_barrier_semaphore()
    for nbr in (left, right):
        pl.semaphore_signal(barrier, inc=1, device_id=nbr,
                            device_id_type=_LOGICAL)
    accA[0] = partial(my_ring, 0)
    accB[0] = partial(my_ring, H)
    pl.semaphore_wait(barrier, 2)
    prevA = [rs_send(subA, 0, 1, rsA_send, rsA_recv, 0, q, right)
             for q in range(Q)]
    prevB = [rs_send(subB, 0, 1, rsB_send, rsB_recv, 0, q, left)
             for q in range(Q)]

    fin = (N_DEV - 1) % NSLOT
    cfA = cw(N_DEV - 1)
    cfB = ccw(N_DEV - 1)
    ag_pend = []
    last = N_DEV - 2
    for s in range(N_DEV - 1):
        rv = (s + 1) % NSLOT
        locA[...] = partial(cw(s + 1), 0)
        locB[...] = partial(ccw(s + 1), H)
        newA, newB = [], []
        for q in range(Q):
            sl = pl.ds(q * HQ, HQ)
            rA = pltpu.make_async_remote_copy(
                src_ref=subA(rv, q), dst_ref=subA(rv, q),
                send_sem=rsA_send.at[s, q], recv_sem=rsA_recv.at[s, q],
                device_id=left, device_id_type=_LOGICAL)
            rA.wait_recv()
            sumA = accA[rv, sl, :] + locA[sl, :]
            if s == last:
                sumA = jnp.maximum(sumA, 0.0)
            accA[rv, sl, :] = sumA
            if s < last:
                if s + 1 >= 3:
                    pl.semaphore_wait(creditA, 1)
                newA.append(rs_send(subA, rv, (s + 2) % NSLOT,
                                    rsA_send, rsA_recv, s + 1, q, right))
            else:
                d = pltpu.make_async_remote_copy(
                    src_ref=subA(rv, q), dst_ref=outA(cfA, q),
                    send_sem=agA_send.at[0, q], recv_sem=agA_recv.at[0, q],
                    device_id=right, device_id_type=_LOGICAL)
                d.start()
                ag_pend.append(d)
            rB = pltpu.make_async_remote_copy(
                src_ref=subB(rv, q), dst_ref=subB(rv, q),
                send_sem=rsB_send.at[s, q], recv_sem=rsB_recv.at[s, q],
                device_id=right, device_id_type=_LOGICAL)
            rB.wait_recv()
            sumB = accB[rv, sl, :] + locB[sl, :]
            if s == last:
                sumB = jnp.maximum(sumB, 0.0)
            accB[rv, sl, :] = sumB
            if s < last:
                if s + 1 >= 3:
                    pl.semaphore_wait(creditB, 1)
                newB.append(rs_send(subB, rv, (s + 2) % NSLOT,
                                    rsB_send, rsB_recv, s + 1, q, left))
            else:
                d = pltpu.make_async_remote_copy(
                    src_ref=subB(rv, q), dst_ref=outB(cfB, q),
                    send_sem=agB_send.at[0, q], recv_sem=agB_recv.at[0, q],
                    device_id=left, device_id_type=_LOGICAL)
                d.start()
                ag_pend.append(d)
        for d in prevA + prevB:
            d.wait_send()
        if s + 3 <= N_DEV - 2:
            for q in range(Q):
                pl.semaphore_signal(creditA, inc=1, device_id=left,
                                    device_id_type=_LOGICAL)
                pl.semaphore_signal(creditB, inc=1, device_id=right,
                                    device_id_type=_LOGICAL)
        prevA, prevB = newA, newB


    cp_own = []
    for q in range(Q):
        c = pltpu.make_async_copy(subA(fin, q), outA(cfA, q),
                                  copy_sems.at[0, q])
        c.start()
        cp_own.append(c)
        c = pltpu.make_async_copy(subB(fin, q), outB(cfB, q),
                                  copy_sems.at[1, q])
        c.start()
        cp_own.append(c)

    for t in range(N_DEV - 1):
        rAc = lax.rem(my_ring - t + 2 * N_DEV, N_DEV)
        rBc = lax.rem(my_ring + t, N_DEV)
        for q in range(Q):
            rA = pltpu.make_async_remote_copy(
                src_ref=outA(rAc, q), dst_ref=outA(rAc, q),
                send_sem=agA_send.at[t, q], recv_sem=agA_recv.at[t, q],
                device_id=left, device_id_type=_LOGICAL)
            rA.wait_recv()
            if t < N_DEV - 2:
                d = pltpu.make_async_remote_copy(
                    src_ref=outA(rAc, q), dst_ref=outA(rAc, q),
                    send_sem=agA_send.at[t + 1, q],
                    recv_sem=agA_recv.at[t + 1, q],
                    device_id=right, device_id_type=_LOGICAL)
                d.start()
                ag_pend.append(d)
            rB = pltpu.make_async_remote_copy(
                src_ref=outB(rBc, q), dst_ref=outB(rBc, q),
                send_sem=agB_send.at[t, q], recv_sem=agB_recv.at[t, q],
                device_id=right, device_id_type=_LOGICAL)
            rB.wait_recv()
            if t < N_DEV - 2:
                d = pltpu.make_async_remote_copy(
                    src_ref=outB(rBc, q), dst_ref=outB(rBc, q),
                    send_sem=agB_send.at[t + 1, q],
                    recv_sem=agB_recv.at[t + 1, q],
                    device_id=left, device_id_type=_LOGICAL)
                d.start()
                ag_pend.append(d)

    for c in cp_own:
        c.wait()
    for d in ag_pend:
        d.wait_send()


def kernel(x, w_mat):
    xbf = x.astype(jnp.bfloat16)
    wbf = w_mat.astype(jnp.bfloat16)
    p = lax.axis_index("i")
    my_ring = jnp.asarray(_MESH_TO_RING, jnp.int32)[p]
    right = jnp.asarray(_RIGHT, jnp.int32)[p]
    left = jnp.asarray(_LEFT, jnp.int32)[p]
    scal = jnp.stack([my_ring, right, left]).astype(jnp.int32)

    return pl.pallas_call(
        _body,
        out_shape=jax.ShapeDtypeStruct((M, N), jnp.float32),
        in_specs=[
            pl.BlockSpec(memory_space=pltpu.SMEM),
            pl.BlockSpec(memory_space=pltpu.VMEM),
            pl.BlockSpec(memory_space=pltpu.VMEM),
        ],
        out_specs=pl.BlockSpec(memory_space=pl.ANY),
        scratch_shapes=[
            pltpu.VMEM((NSLOT, H, N), jnp.float32),
            pltpu.VMEM((NSLOT, H, N), jnp.float32),
            pltpu.VMEM((H, N), jnp.float32),
            pltpu.VMEM((H, N), jnp.float32),
            pltpu.SemaphoreType.DMA((2, Q)),
            pltpu.SemaphoreType.DMA((N_DEV - 1, Q)),
            pltpu.SemaphoreType.DMA((N_DEV - 1, Q)),
            pltpu.SemaphoreType.DMA((N_DEV - 1, Q)),
            pltpu.SemaphoreType.DMA((N_DEV - 1, Q)),
            pltpu.SemaphoreType.DMA((N_DEV - 1, Q)),
            pltpu.SemaphoreType.DMA((N_DEV - 1, Q)),
            pltpu.SemaphoreType.DMA((N_DEV - 1, Q)),
            pltpu.SemaphoreType.DMA((N_DEV - 1, Q)),
            pltpu.SemaphoreType.REGULAR,
            pltpu.SemaphoreType.REGULAR,
        ],
        compiler_params=pltpu.CompilerParams(
            collective_id=0, vmem_limit_bytes=100 * 1024 * 1024),
    )(scal, xbf, wbf)


# device time: 1135908 ns/iter; 1.2952x vs baseline; 1.2952x over previous
import jax
import jax.numpy as jnp
from jax import lax
from jax.experimental import pallas as pl
from jax.experimental.pallas import tpu as pltpu

N_DEV = 16
M, N = 8192, 4096
MC = M // N_DEV
H = MC // 2
Q = 4
HQ = H // Q
NSLOT = 4

_PLANE = {(0, 0): 0, (1, 0): 1, (1, 1): 2, (0, 1): 3}
_RING_COORDS = (
    [(0, 0, z) for z in range(4)]
    + [(1, 0, z) for z in range(3, -1, -1)]
    + [(1, 1, z) for z in range(4)]
    + [(0, 1, z) for z in range(3, -1, -1)]
)
_RING_MESH = [4 * z + _PLANE[(x, y)] for (x, y, z) in _RING_COORDS]
_MESH_TO_RING = [0] * N_DEV
for _r, _p in enumerate(_RING_MESH):
    _MESH_TO_RING[_p] = _r
_RIGHT = [_RING_MESH[(_MESH_TO_RING[p] + 1) % N_DEV] for p in range(N_DEV)]
_LEFT = [_RING_MESH[(_MESH_TO_RING[p] - 1) % N_DEV] for p in range(N_DEV)]

_F32 = jnp.float32
_LOGICAL = pl.DeviceIdType.LOGICAL


def _body(scal_ref, xbf, wbf, out_ref, accA, accB, locA, locB, finA, finB,
          copy_sems, rsA_send, rsA_recv, rsB_send, rsB_recv,
          agA_send, agA_recv, agB_send, agB_recv, creditA, creditB):
    my_ring = scal_ref[0]
    right = scal_ref[1]
    left = scal_ref[2]

    def cw(k):
        return lax.rem(my_ring - k + 2 * N_DEV, N_DEV)

    def ccw(k):
        return lax.rem(my_ring + k, N_DEV)

    def subA(slot, q):
        return accA.at[slot, pl.ds(q * HQ, HQ), :]

    def subB(slot, q):
        return accB.at[slot, pl.ds(q * HQ, HQ), :]

    def outA(c, q):
        return out_ref.at[pl.ds(c * MC + q * HQ, HQ), :]

    def outB(c, q):
        return out_ref.at[pl.ds(c * MC + H + q * HQ, HQ), :]

    def partial(c, half_off):
        xs = xbf[pl.ds(c * MC + half_off, H), :]
        return jnp.dot(xs, wbf[...], preferred_element_type=_F32)

    def rs_send(ring_sub, slot_src, slot_dst, sems_s, sems_r, s, q, dev):
        d = pltpu.make_async_remote_copy(
            src_ref=ring_sub(slot_src, q), dst_ref=ring_sub(slot_dst, q),
            send_sem=sems_s.at[s, q], recv_sem=sems_r.at[s, q],
            device_id=dev, device_id_type=_LOGICAL)
        d.start()
        return d

    barrier = pltpu.get_barrier_semaphore()
    for nbr in (left, right):
        pl.semaphore_signal(barrier, inc=1, device_id=nbr,
                            device_id_type=_LOGICAL)
    accA[0] = partial(my_ring, 0).astype(jnp.bfloat16)
    accB[0] = partial(my_ring, H).astype(jnp.bfloat16)
    pl.semaphore_wait(barrier, 2)
    prevA = [rs_send(subA, 0, 1, rsA_send, rsA_recv, 0, q, right)
             for q in range(Q)]
    prevB = [rs_send(subB, 0, 1, rsB_send, rsB_recv, 0, q, left)
             for q in range(Q)]

    cfA = cw(N_DEV - 1)
    cfB = ccw(N_DEV - 1)
    ag_pend = []
    last = N_DEV - 2
    for s in range(N_DEV - 1):
        rv = (s + 1) % NSLOT
        locA[...] = partial(cw(s + 1), 0)
        locB[...] = partial(ccw(s + 1), H)
        newA, newB = [], []
        for q in range(Q):
            sl = pl.ds(q * HQ, HQ)
            rA = pltpu.make_async_remote_copy(
                src_ref=subA(rv, q), dst_ref=subA(rv, q),
                send_sem=rsA_send.at[s, q], recv_sem=rsA_recv.at[s, q],
                device_id=left, device_id_type=_LOGICAL)
            rA.wait_recv()
            sumA = accA[rv, sl, :].astype(_F32) + locA[sl, :]
            if s < last:
                accA[rv, sl, :] = sumA.astype(jnp.bfloat16)
                if s + 1 >= 3:
                    pl.semaphore_wait(creditA, 1)
                newA.append(rs_send(subA, rv, (s + 2) % NSLOT,
                                    rsA_send, rsA_recv, s + 1, q, right))
            else:
                finA[sl, :] = jnp.maximum(sumA, 0.0)
                d = pltpu.make_async_remote_copy(
                    src_ref=finA.at[sl, :], dst_ref=outA(cfA, q),
                    send_sem=agA_send.at[0, q], recv_sem=agA_recv.at[0, q],
                    device_id=right, device_id_type=_LOGICAL)
                d.start()
                ag_pend.append(d)
            rB = pltpu.make_async_remote_copy(
                src_ref=subB(rv, q), dst_ref=subB(rv, q),
                send_sem=rsB_send.at[s, q], recv_sem=rsB_recv.at[s, q],
                device_id=right, device_id_type=_LOGICAL)
            rB.wait_recv()
            sumB = accB[rv, sl, :].astype(_F32) + locB[sl, :]
            if s < last:
                accB[rv, sl, :] = sumB.astype(jnp.bfloat16)
                if s + 1 >= 3:
                    pl.semaphore_wait(creditB, 1)
                newB.append(rs_send(subB, rv, (s + 2) % NSLOT,
                                    rsB_send, rsB_recv, s + 1, q, left))
            else:
                finB[sl, :] = jnp.maximum(sumB, 0.0)
                d = pltpu.make_async_remote_copy(
                    src_ref=finB.at[sl, :], dst_ref=outB(cfB, q),
                    send_sem=agB_send.at[0, q], recv_sem=agB_recv.at[0, q],
                    device_id=left, device_id_type=_LOGICAL)
                d.start()
                ag_pend.append(d)
        for d in prevA + prevB:
            d.wait_send()
        if s + 3 <= N_DEV - 2:
            for q in range(Q):
                pl.semaphore_signal(creditA, inc=1, device_id=left,
                                    device_id_type=_LOGICAL)
                pl.semaphore_signal(creditB, inc=1, device_id=right,
                                    device_id_type=_LOGICAL)
        prevA, prevB = newA, newB


    cp_own = []
    for q in range(Q):
        sl = pl.ds(q * HQ, HQ)
        c = pltpu.make_async_copy(finA.at[sl, :], outA(cfA, q),
                                  copy_sems.at[0, q])
        c.start()
        cp_own.append(c)
        c = pltpu.make_async_copy(finB.at[sl, :], outB(cfB, q),
                                  copy_sems.at[1, q])
        c.start()
        cp_own.append(c)

    for t in range(N_DEV - 1):
        rAc = lax.rem(my_ring - t + 2 * N_DEV, N_DEV)
        rBc = lax.rem(my_ring + t, N_DEV)
        for q in range(Q):
            rA = pltpu.make_async_remote_copy(
                src_ref=outA(rAc, q), dst_ref=outA(rAc, q),
                send_sem=agA_send.at[t, q], recv_sem=agA_recv.at[t, q],
                device_id=left, device_id_type=_LOGICAL)
            rA.wait_recv()
            if t < N_DEV - 2:
                d = pltpu.make_async_remote_copy(
                    src_ref=outA(rAc, q), dst_ref=outA(rAc, q),
                    send_sem=agA_send.at[t + 1, q],
                    recv_sem=agA_recv.at[t + 1, q],
                    device_id=right, device_id_type=_LOGICAL)
                d.start()
                ag_pend.append(d)
            rB = pltpu.make_async_remote_copy(
                src_ref=outB(rBc, q), dst_ref=outB(rBc, q),
                send_sem=agB_send.at[t, q], recv_sem=agB_recv.at[t, q],
                device_id=right, device_id_type=_LOGICAL)
            rB.wait_recv()
            if t < N_DEV - 2:
                d = pltpu.make_async_remote_copy(
                    src_ref=outB(rBc, q), dst_ref=outB(rBc, q),
                    send_sem=agB_send.at[t + 1, q],
                    recv_sem=agB_recv.at[t + 1, q],
                    device_id=left, device_id_type=_LOGICAL)
                d.start()
                ag_pend.append(d)

    for c in cp_own:
        c.wait()
    for d in ag_pend:
        d.wait_send()


def kernel(x, w_mat):
    xbf = x.astype(jnp.bfloat16)
    wbf = w_mat.astype(jnp.bfloat16)
    p = lax.axis_index("i")
    my_ring = jnp.asarray(_MESH_TO_RING, jnp.int32)[p]
    right = jnp.asarray(_RIGHT, jnp.int32)[p]
    left = jnp.asarray(_LEFT, jnp.int32)[p]
    scal = jnp.stack([my_ring, right, left]).astype(jnp.int32)

    return pl.pallas_call(
        _body,
        out_shape=jax.ShapeDtypeStruct((M, N), jnp.float32),
        in_specs=[
            pl.BlockSpec(memory_space=pltpu.SMEM),
            pl.BlockSpec(memory_space=pltpu.VMEM),
            pl.BlockSpec(memory_space=pltpu.VMEM),
        ],
        out_specs=pl.BlockSpec(memory_space=pl.ANY),
        scratch_shapes=[
            pltpu.VMEM((NSLOT, H, N), jnp.bfloat16),
            pltpu.VMEM((NSLOT, H, N), jnp.bfloat16),
            pltpu.VMEM((H, N), jnp.float32),
            pltpu.VMEM((H, N), jnp.float32),
            pltpu.VMEM((H, N), jnp.float32),
            pltpu.VMEM((H, N), jnp.float32),
            pltpu.SemaphoreType.DMA((2, Q)),
            pltpu.SemaphoreType.DMA((N_DEV - 1, Q)),
            pltpu.SemaphoreType.DMA((N_DEV - 1, Q)),
            pltpu.SemaphoreType.DMA((N_DEV - 1, Q)),
            pltpu.SemaphoreType.DMA((N_DEV - 1, Q)),
            pltpu.SemaphoreType.DMA((N_DEV - 1, Q)),
            pltpu.SemaphoreType.DMA((N_DEV - 1, Q)),
            pltpu.SemaphoreType.DMA((N_DEV - 1, Q)),
            pltpu.SemaphoreType.DMA((N_DEV - 1, Q)),
            pltpu.SemaphoreType.REGULAR,
            pltpu.SemaphoreType.REGULAR,
        ],
        compiler_params=pltpu.CompilerParams(
            collective_id=0, vmem_limit_bytes=100 * 1024 * 1024),
    )(scal, xbf, wbf)
